# baseline (speedup 1.0000x reference)
import functools

import jax
import jax.numpy as jnp
from jax import lax
from jax.experimental import pallas as pl
from jax.experimental.pallas import tpu as pltpu
from jax.experimental.pallas import tpu_sc as plsc

_NC = 2
_NS = 16
_NW = _NC * _NS
_L = 16


def _prep_body(t_ref, cls_ref, iou_ref):
    cls = t_ref[:, 0:1]
    iou = t_ref[:, 1:2].astype(jnp.float32)
    cls_ref[...] = cls
    iou_ref[...] = jnp.where(cls >= 1, iou, -1.0)


def _sc_gather(cls_flat, logits_flat, n, c):
    rows_w = n // _NW
    chunks = rows_w // 128
    mesh = plsc.VectorSubcoreMesh(core_axis_name="c", subcore_axis_name="s",
                                  num_cores=_NC, num_subcores=_NS)

    @functools.partial(
        pl.kernel, mesh=mesh,
        out_type=jax.ShapeDtypeStruct((n,), jnp.float32),
        scratch_types=[
            pltpu.VMEM((rows_w,), jnp.int32),
            pltpu.VMEM((rows_w,), jnp.int32),
            pltpu.VMEM((rows_w,), jnp.float32),
            pltpu.SemaphoreType.DMA,
        ],
    )
    def k(cls_hbm, logits_hbm, g_hbm, cls_v, idx_v, g_v, sem):
        wid = lax.axis_index("s") * _NC + lax.axis_index("c")
        base = wid * rows_w
        pltpu.sync_copy(cls_hbm.at[pl.ds(base, rows_w)], cls_v)

        def chunk_body(j, carry):
            for b in range(8):
                off = j * 128 + b * 16
                lane = lax.iota(jnp.int32, _L)
                v = cls_v[pl.ds(off, _L)]
                safe = jnp.clip(v - 1, 0, c - 1)
                idx_v[pl.ds(off, _L)] = (base + off + lane) * c + safe
            pltpu.async_copy(
                logits_hbm.at[idx_v.at[pl.ds(j * 128, 128)]],
                g_v.at[pl.ds(j * 128, 128)], sem)
            return carry

        lax.fori_loop(0, chunks, chunk_body, 0)
        pltpu.make_async_copy(logits_hbm.at[pl.ds(0, rows_w)], g_v, sem).wait()
        pltpu.sync_copy(g_v, g_hbm.at[pl.ds(base, rows_w)])

    return k(cls_flat, logits_flat)


def _dense_body(x_ref, out_ref):
    i = pl.program_id(0)
    l = x_ref[:]
    th = jnp.tanh(0.5 * l)
    q = jnp.maximum(0.5 - 0.5 * th, 1e-37)
    s = jnp.sum(jnp.log(q) - q)

    @pl.when(i == 0)
    def _():
        out_ref[...] = jnp.zeros_like(out_ref)

    out_ref[...] += s.reshape(1, 1)


def _corr_body(s0_ref, g_ref, iou_ref, out_ref, *, count):
    ie = iou_ref[:]
    m = jnp.max(ie)
    lab = jnp.maximum(ie, 0.0) * (1.0 / m)
    l = g_ref[:]
    th = jnp.tanh(0.5 * l)
    q = jnp.maximum(0.5 - 0.5 * th, 1e-37)
    p = 1.0 - q
    sp = -jnp.log(q)
    c_neg = 0.75 * lab * (1.0 - sp)
    c_pos = 0.25 * lab * (sp - l - 1.0) + p - 0.75 * sp
    corr = jnp.where(p <= lab, c_pos, c_neg)
    out_ref[...] = (-0.75) * (s0_ref[...] + count) + jnp.sum(corr).reshape(1, 1)


def kernel(logits, targets):
    n, c = logits.shape
    logits_flat = logits.reshape(-1)

    pb = 4096
    cls, iou_enc = pl.pallas_call(
        _prep_body,
        grid=(n // pb,),
        in_specs=[pl.BlockSpec((pb, 2), lambda i: (i, 0))],
        out_specs=[pl.BlockSpec((pb, 1), lambda i: (i, 0)),
                   pl.BlockSpec((pb, 1), lambda i: (i, 0))],
        out_shape=[jax.ShapeDtypeStruct((n, 1), jnp.int32),
                   jax.ShapeDtypeStruct((n, 1), jnp.float32)],
    )(targets)

    g = _sc_gather(cls.reshape(n), logits_flat, n, c)

    rows = 1024
    wide = (n * c) // 32768
    xf = logits.reshape(32768, wide)
    s0 = pl.pallas_call(
        _dense_body,
        grid=(32768 // rows,),
        in_specs=[pl.BlockSpec((rows, wide), lambda i: (i, 0))],
        out_specs=pl.BlockSpec((1, 1), lambda i: (0, 0)),
        out_shape=jax.ShapeDtypeStruct((1, 1), jnp.float32),
    )(xf)

    out = pl.pallas_call(
        functools.partial(_corr_body, count=float(n * c)),
        in_specs=[
            pl.BlockSpec((1, 1), lambda: (0, 0)),
            pl.BlockSpec((n // 128, 128), lambda: (0, 0)),
            pl.BlockSpec((n // 128, 128), lambda: (0, 0)),
        ],
        out_specs=pl.BlockSpec((1, 1), lambda: (0, 0)),
        out_shape=jax.ShapeDtypeStruct((1, 1), jnp.float32),
    )(s0, g.reshape(n // 128, 128), iou_enc.reshape(n // 128, 128))
    return out[0, 0]


# speedup vs baseline: 1.5875x; 1.5875x over previous
import functools

import jax
import jax.numpy as jnp
from jax import lax
from jax.experimental import pallas as pl
from jax.experimental.pallas import tpu as pltpu
from jax.experimental.pallas import tpu_sc as plsc

_NC = 2
_NS = 16
_NW = _NC * _NS
_L = 16


def _sc_label_max(targets_flat, n):
    words_w = 2 * (n // _NW)
    mesh = plsc.VectorSubcoreMesh(core_axis_name="c", subcore_axis_name="s",
                                  num_cores=_NC, num_subcores=_NS)

    @functools.partial(
        pl.kernel, mesh=mesh,
        out_type=jax.ShapeDtypeStruct((_NW * _L,), jnp.float32),
        scratch_types=[
            pltpu.VMEM((words_w + 8,), jnp.int32),
            pltpu.VMEM((_L,), jnp.float32),
            pltpu.VMEM((_L,), jnp.int32),
        ],
    )
    def k(t_hbm, out_hbm, t_v, m_v, mx_v):
        wid = lax.axis_index("s") * _NC + lax.axis_index("c")
        base = wid * words_w
        pltpu.sync_copy(t_hbm.at[pl.ds(base, words_w)],
                        t_v.at[pl.ds(0, words_w)])
        lane = lax.iota(jnp.int32, _L)
        even = (lane % 2) == 0

        mx_v[...] = jnp.zeros((_L,), jnp.int32)

        def body(j, carry):
            off = j * _L
            v_cls = t_v[pl.ds(off, _L)]
            v_iou = t_v[pl.ds(off + 1, _L)]
            val = jnp.where(even & (v_cls >= 1), v_iou, 0)
            mx_v[...] = jnp.maximum(mx_v[...], val)
            return carry

        lax.fori_loop(0, words_w // _L, body, 0)
        m_v[...] = mx_v[...].astype(jnp.float32)
        pltpu.sync_copy(m_v, out_hbm.at[pl.ds(wid * _L, _L)])

    return k(targets_flat)


def _loss_body(m_ref, x_ref, t_ref, out_ref, invm_ref):
    i = pl.program_id(0)

    @pl.when(i == 0)
    def _():
        invm_ref[0, 0] = 1.0 / jnp.max(m_ref[:])
        out_ref[...] = jnp.zeros_like(out_ref)

    invm = invm_ref[0, 0]
    l = x_ref[:]
    b, c = l.shape
    cls = t_ref[:, 0:1]
    iou = t_ref[:, 1:2].astype(jnp.float32)
    col = lax.broadcasted_iota(jnp.int32, (b, c), 1)
    lab = jnp.where(col == cls - 1, iou * invm, 0.0)

    th = jnp.tanh(0.5 * l)
    q = jnp.maximum(0.5 - 0.5 * th, 1e-37)
    p = 1.0 - q
    sp = -jnp.log(q)
    term_neg = (sp * (1.0 - lab) + lab - p) * 0.75
    term_pos = ((sp - l) * lab + p - lab) * 0.25
    t = jnp.where(p <= lab, term_pos, term_neg)
    out_ref[...] += jnp.sum(t).reshape(1, 1)


def kernel(logits, targets):
    n, c = logits.shape

    m512 = _sc_label_max(targets.reshape(-1), n)

    bb = 4096
    out = pl.pallas_call(
        _loss_body,
        grid=(n // bb,),
        in_specs=[
            pl.BlockSpec((4, 128), lambda i: (0, 0)),
            pl.BlockSpec((bb, c), lambda i: (i, 0)),
            pl.BlockSpec((bb, 2), lambda i: (i, 0)),
        ],
        out_specs=pl.BlockSpec((1, 1), lambda i: (0, 0)),
        out_shape=jax.ShapeDtypeStruct((1, 1), jnp.float32),
        scratch_shapes=[pltpu.SMEM((1, 1), jnp.float32)],
    )(m512.reshape(4, 128), logits, targets)
    return out[0, 0]
